# Initial kernel scaffold; baseline (speedup 1.0000x reference)
import jax
import jax.numpy as jnp
from jax.experimental import pallas as pl


def kernel(anchor, key, support):
    raise NotImplementedError("write your pallas kernel here")


# speedup vs baseline: 170.8903x; 170.8903x over previous
import functools

import jax
import jax.numpy as jnp
from jax import lax
from jax.experimental import pallas as pl
from jax.experimental.pallas import tpu as pltpu
from jax.experimental.pallas import tpu_sc as plsc

B = 1024
F = 128
N = 100000
BN = 2048
NB = (N + BN - 1) // BN

IBIG = 2**31 - 1


def _lex_lt(av, ai, bv, bi):
    return (av < bv) | ((av == bv) & (ai < bi))


def _top2_body(supp_ref, key_ref, out_ref, v1s, i1s, v2s, i2s):
    j = pl.program_id(0)
    sims = lax.dot_general(
        supp_ref[...], key_ref[...],
        dimension_numbers=(((0,), (1,)), ((), ())),
        preferred_element_type=jnp.float32)
    rowid = lax.broadcasted_iota(jnp.int32, (BN, B), 0) + j * BN
    sims = jnp.where(rowid < N, sims, jnp.inf)

    m1 = jnp.min(sims, axis=0, keepdims=True)
    i1 = jnp.min(jnp.where(sims <= m1, rowid, IBIG), axis=0, keepdims=True)
    sims2 = jnp.where(rowid == i1, jnp.inf, sims)
    m2 = jnp.min(sims2, axis=0, keepdims=True)
    i2 = jnp.min(jnp.where(sims2 <= m2, rowid, IBIG), axis=0, keepdims=True)

    @pl.when(j == 0)
    def _init():
        v1s[...] = m1
        i1s[...] = i1
        v2s[...] = m2
        i2s[...] = i2

    @pl.when(j > 0)
    def _merge():
        r1v, r1i = v1s[...], i1s[...]
        r2v, r2i = v2s[...], i2s[...]
        take_r = _lex_lt(r1v, r1i, m1, i1)
        n1v = jnp.where(take_r, r1v, m1)
        n1i = jnp.where(take_r, r1i, i1)
        lv = jnp.where(take_r, m1, r1v)
        li = jnp.where(take_r, i1, r1i)
        wv = jnp.where(take_r, r2v, m2)
        wi = jnp.where(take_r, r2i, i2)
        t2 = _lex_lt(lv, li, wv, wi)
        v1s[...] = n1v
        i1s[...] = n1i
        v2s[...] = jnp.where(t2, lv, wv)
        i2s[...] = jnp.where(t2, li, wi)

    out_ref[...] = i2s[...]


def _neighbor_idx(key, support):
    return pl.pallas_call(
        _top2_body,
        grid=(NB,),
        in_specs=[
            pl.BlockSpec((F, BN), lambda j: (0, j)),
            pl.BlockSpec((B, F), lambda j: (0, 0)),
        ],
        out_specs=pl.BlockSpec((1, B), lambda j: (0, 0)),
        out_shape=jax.ShapeDtypeStruct((1, B), jnp.int32),
        scratch_shapes=[
            pltpu.VMEM((1, B), jnp.float32),
            pltpu.VMEM((1, B), jnp.int32),
            pltpu.VMEM((1, B), jnp.float32),
            pltpu.VMEM((1, B), jnp.int32),
        ],
    )(support, key)


_NC, _NS, _L = 2, 16, 16
_NW = _NC * _NS
_BPW = B // _NW


def _sc_gather_dot(support_flat, idx, anchor_t):
    mesh = plsc.VectorSubcoreMesh(core_axis_name="c", subcore_axis_name="s")

    nchunk = F * _BPW // 128
    fpc = 128 // _BPW

    @functools.partial(
        pl.kernel,
        mesh=mesh,
        out_type=jax.ShapeDtypeStruct((B,), jnp.float32),
        scratch_types=[
            pltpu.VMEM((_BPW,), jnp.int32),
            pltpu.VMEM((nchunk, 128), jnp.int32),
            pltpu.VMEM((nchunk, 128), jnp.float32),
            pltpu.VMEM((F, _BPW), jnp.float32),
            pltpu.VMEM((_BPW,), jnp.float32),
            pltpu.SemaphoreType.DMA,
        ],
    )
    def k(supp_hbm, idx_hbm, anc_hbm, out_hbm, idx_v, fidx_v, gath_v, anc_v,
          out_v, sem):
        wid = lax.axis_index("s") * _NC + lax.axis_index("c")
        base = wid * _BPW
        pltpu.sync_copy(idx_hbm.at[pl.ds(base, _BPW)], idx_v)
        pltpu.sync_copy(anc_hbm.at[wid], anc_v)
        ia = idx_v[pl.ds(0, _L)]
        ib = idx_v[pl.ds(_L, _L)]

        def fill(c, _):
            for q in range(fpc):
                off = (fpc * c + q) * N
                fidx_v[c, pl.ds(q * _BPW, _L)] = ia + off
                fidx_v[c, pl.ds(q * _BPW + _L, _L)] = ib + off
            return 0

        lax.fori_loop(0, nchunk, fill, 0)

        def fire(c, _):
            pltpu.async_copy(supp_hbm.at[fidx_v.at[c]], gath_v.at[c], sem)
            return 0

        lax.fori_loop(0, nchunk, fire, 0)

        def drain(c, _):
            pltpu.make_async_copy(
                supp_hbm.at[fidx_v.at[c]], gath_v.at[c], sem).wait()
            return 0

        lax.fori_loop(0, nchunk, drain, 0)

        def acc(c, carry):
            a0, a1 = carry
            for q in range(fpc):
                f = fpc * c + q
                a0 = a0 + anc_v[f, pl.ds(0, _L)] * gath_v[c, pl.ds(q * _BPW, _L)]
                a1 = a1 + anc_v[f, pl.ds(_L, _L)] * gath_v[c, pl.ds(q * _BPW + _L, _L)]
            return (a0, a1)

        zero = jnp.zeros((_L,), jnp.float32)
        a0, a1 = lax.fori_loop(0, nchunk, acc, (zero, zero))
        out_v[pl.ds(0, _L)] = a0
        out_v[pl.ds(_L, _L)] = a1
        pltpu.sync_copy(out_v, out_hbm.at[pl.ds(base, _BPW)])

    return k(support_flat, idx, anchor_t)


def kernel(anchor, key, support):
    idx = _neighbor_idx(key, support).reshape(B)
    support_flat = support.reshape(F * N)
    anchor_t = anchor.T.reshape(F, _NW, _BPW).transpose(1, 0, 2)
    out = _sc_gather_dot(support_flat, idx, anchor_t)
    return out.reshape(B, 1)
